# baseline (speedup 1.0000x reference)
import functools

import jax
import jax.numpy as jnp
from jax import lax
from jax.experimental import pallas as pl
from jax.experimental.pallas import tpu as pltpu
from jax.experimental.pallas import tpu_sc as plsc

N = 20000
TPAD = 20480
TROWS = TPAD // 128

MTC = 12800
GTC = MTC // 32

NSC = N - MTC
NW = 32
SBLK = 8
NSC_PAD = ((NSC + NW * SBLK - 1) // (NW * SBLK)) * (NW * SBLK)
SRC_PER_W = NSC_PAD // NW
UNROLL = 2
NTV = N // 16
CB_T = 640
NCOL = TPAD


def _tc_sweep(sx, sy, sz, sq, tx, ty, tz, tq):
    def body(sx_r, sy_r, sz_r, sq_r, tx_r, ty_r, tz_r, tq_r,
             rowsum_r, colpart_r, colacc_r):
        i = pl.program_id(0)

        @pl.when(i == 0)
        def _():
            def ini(t, carry):
                colacc_r[:, pl.ds(t * 128, 128)] = jnp.full(
                    (8, 128), 3.0e38, jnp.float32)
                return carry
            lax.fori_loop(0, TROWS, ini, 0)

        sxo = [sx_r[o] for o in range(4)]
        syo = [sy_r[o] for o in range(4)]
        szo = [sz_r[o] for o in range(4)]
        sqo = [sq_r[o] for o in range(4)]

        rowinit = tuple(jnp.full((8, 128), 3.0e38, jnp.float32)
                        for _ in range(4))

        def tile(t, rowaccs):
            rowaccs = list(rowaccs)
            txv = tx_r[t]
            tyv = ty_r[t]
            tzv = tz_r[t]
            tqv = tq_r[t]
            slab = colacc_r[:, pl.ds(t * 128, 128)]
            for o in range(4):
                e = tqv + txv * sxo[o] + tyv * syo[o] + tzv * szo[o]
                rowaccs[o] = jnp.minimum(rowaccs[o], e)
                slab = jnp.minimum(slab, e + sqo[o])
            colacc_r[:, pl.ds(t * 128, 128)] = slab
            return tuple(rowaccs)

        rowaccs = lax.fori_loop(0, TROWS, tile, rowinit)

        blocksum = jnp.float32(0.0)
        for o in range(4):
            rmin = jnp.min(rowaccs[o], axis=1, keepdims=True)
            blocksum = blocksum + jnp.sum(rmin + sqo[o])
        blocksum = blocksum.reshape(1, 1)

        @pl.when(i == 0)
        def _():
            rowsum_r[...] = blocksum

        @pl.when(i > 0)
        def _():
            rowsum_r[...] = rowsum_r[...] + blocksum

        @pl.when(i == GTC - 1)
        def _():
            def fin(t, carry):
                slab = colacc_r[:, pl.ds(t * 128, 128)]
                colpart_r[t] = jnp.min(slab, axis=0, keepdims=True)
                return carry
            lax.fori_loop(0, TROWS, fin, 0)

    return pl.pallas_call(
        body,
        grid=(GTC,),
        in_specs=[
            pl.BlockSpec((4, 8, 1), lambda i: (i, 0, 0)),
            pl.BlockSpec((4, 8, 1), lambda i: (i, 0, 0)),
            pl.BlockSpec((4, 8, 1), lambda i: (i, 0, 0)),
            pl.BlockSpec((4, 8, 1), lambda i: (i, 0, 0)),
            pl.BlockSpec((TROWS, 1, 128), lambda i: (0, 0, 0)),
            pl.BlockSpec((TROWS, 1, 128), lambda i: (0, 0, 0)),
            pl.BlockSpec((TROWS, 1, 128), lambda i: (0, 0, 0)),
            pl.BlockSpec((TROWS, 1, 128), lambda i: (0, 0, 0)),
        ],
        out_specs=[
            pl.BlockSpec((1, 1), lambda i: (0, 0)),
            pl.BlockSpec((TROWS, 1, 128), lambda i: (0, 0, 0)),
        ],
        out_shape=[
            jax.ShapeDtypeStruct((1, 1), jnp.float32),
            jax.ShapeDtypeStruct((TROWS, 1, 128), jnp.float32),
        ],
        scratch_shapes=[pltpu.VMEM((8, TPAD), jnp.float32)],
    )(sx, sy, sz, sq, tx, ty, tz, tq)


def _sc_sweep(src_x, src_y, src_z, src_q, src_m, tgt_x, tgt_y, tgt_z, tgt_q):
    mesh = plsc.VectorSubcoreMesh(core_axis_name="c", subcore_axis_name="s")

    @functools.partial(
        pl.kernel,
        mesh=mesh,
        out_type=(jax.ShapeDtypeStruct((NW, 16), jnp.float32),
                  jax.ShapeDtypeStruct((NW, NCOL), jnp.float32)),
        compiler_params=pltpu.CompilerParams(needs_layout_passes=False),
        scratch_types=[
            pltpu.VMEM((N,), jnp.float32),
            pltpu.VMEM((N,), jnp.float32),
            pltpu.VMEM((N,), jnp.float32),
            pltpu.VMEM((N,), jnp.float32),
            pltpu.VMEM((NCOL,), jnp.float32),
            pltpu.VMEM((SRC_PER_W + 16,), jnp.float32),
            pltpu.VMEM((SRC_PER_W + 16,), jnp.float32),
            pltpu.VMEM((SRC_PER_W + 16,), jnp.float32),
            pltpu.VMEM((SRC_PER_W + 16,), jnp.float32),
            pltpu.VMEM((SRC_PER_W + 16,), jnp.float32),
            pltpu.VMEM((16,), jnp.float32),
        ],
    )
    def body(src_x_h, src_y_h, src_z_h, src_q_h, src_m_h,
             tgt_x_h, tgt_y_h, tgt_z_h, tgt_q_h,
             rows_h, cols_h,
             tx_v, ty_v, tz_v, tq_v, col_v,
             sx_v, sy_v, sz_v, sq_v, sm_v, out_v):
        c = lax.axis_index("c")
        s = lax.axis_index("s")
        wid = c * 16 + s
        base = wid * SRC_PER_W

        pltpu.sync_copy(tgt_x_h, tx_v)
        pltpu.sync_copy(tgt_y_h, ty_v)
        pltpu.sync_copy(tgt_z_h, tz_v)
        pltpu.sync_copy(tgt_q_h, tq_v)
        pltpu.sync_copy(src_x_h.at[pl.ds(base, SRC_PER_W)],
                        sx_v.at[pl.ds(0, SRC_PER_W)])
        pltpu.sync_copy(src_y_h.at[pl.ds(base, SRC_PER_W)],
                        sy_v.at[pl.ds(0, SRC_PER_W)])
        pltpu.sync_copy(src_z_h.at[pl.ds(base, SRC_PER_W)],
                        sz_v.at[pl.ds(0, SRC_PER_W)])
        pltpu.sync_copy(src_q_h.at[pl.ds(base, SRC_PER_W)],
                        sq_v.at[pl.ds(0, SRC_PER_W)])
        pltpu.sync_copy(src_m_h.at[pl.ds(base, SRC_PER_W)],
                        sm_v.at[pl.ds(0, SRC_PER_W)])

        big = jnp.full((16,), 3.0e38, jnp.float32)

        def col_init(j, carry):
            col_v[pl.ds(j * 16, 16)] = big
            return carry

        lax.fori_loop(0, NTV, col_init, jnp.float32(0.0))

        zeros16 = jnp.zeros((16,), jnp.float32)

        def col_tail_init(j, carry):
            col_v[pl.ds(NTV * 16 + j * 16, 16)] = zeros16
            return carry

        lax.fori_loop(0, (NCOL - N) // 16, col_tail_init, jnp.float32(0.0))

        def src_block(b, psum):
            i0 = b * SBLK
            vx = sx_v[pl.ds(i0, 16)]
            vy = sy_v[pl.ds(i0, 16)]
            vz = sz_v[pl.ds(i0, 16)]
            vq = sq_v[pl.ds(i0, 16)]
            bx, by, bz, bq = [], [], [], []
            for k in range(SBLK):
                bx.append(jnp.full((16,), vx[k], jnp.float32))
                by.append(jnp.full((16,), vy[k], jnp.float32))
                bz.append(jnp.full((16,), vz[k], jnp.float32))
                bq.append(jnp.full((16,), vq[k], jnp.float32))

            init = tuple(big for _ in range(SBLK))

            def tgt_iter(j, accs):
                accs = list(accs)
                o = j * (16 * UNROLL)
                for u in range(UNROLL):
                    off = o + u * 16
                    tx = tx_v[pl.ds(off, 16)]
                    ty = ty_v[pl.ds(off, 16)]
                    tz = tz_v[pl.ds(off, 16)]
                    tq = tq_v[pl.ds(off, 16)]
                    cmin = None
                    for k in range(SBLK):
                        e = tq + bx[k] * tx + by[k] * ty + bz[k] * tz
                        accs[k] = jnp.minimum(accs[k], e)
                        d = e + bq[k]
                        cmin = d if cmin is None else jnp.minimum(cmin, d)
                    col_v[pl.ds(off, 16)] = jnp.minimum(
                        col_v[pl.ds(off, 16)], cmin)
                return tuple(accs)

            accs = lax.fori_loop(0, NTV // UNROLL, tgt_iter, init)

            mv = sm_v[pl.ds(i0, 16)]
            for k in range(SBLK):
                psum = psum + (jnp.min(accs[k]) + vq[k]) * mv[k]
            return psum

        psum = lax.fori_loop(0, SRC_PER_W // SBLK, src_block,
                             jnp.float32(0.0))

        lane = lax.broadcasted_iota(jnp.int32, (16,), 0)
        out_v[...] = jnp.where(lane == 0, psum, 0.0)
        pltpu.sync_copy(out_v, rows_h.at[wid])
        pltpu.sync_copy(col_v, cols_h.at[wid])

    return body(src_x, src_y, src_z, src_q, src_m,
                tgt_x, tgt_y, tgt_z, tgt_q)


def _sc_combine(cols, coltc):
    mesh = plsc.VectorSubcoreMesh(core_axis_name="c", subcore_axis_name="s")

    @functools.partial(
        pl.kernel,
        mesh=mesh,
        out_type=jax.ShapeDtypeStruct((NW, 16), jnp.float32),
        compiler_params=pltpu.CompilerParams(needs_layout_passes=False),
        scratch_types=[
            pltpu.VMEM((NW, CB_T), jnp.float32),
            pltpu.VMEM((CB_T,), jnp.float32),
            pltpu.VMEM((16,), jnp.float32),
        ],
    )
    def body(cols_h, coltc_h, out_h, rows_v, tc_v, out_v):
        c = lax.axis_index("c")
        s = lax.axis_index("s")
        wid = c * 16 + s
        base = wid * CB_T

        pltpu.sync_copy(cols_h.at[:, pl.ds(base, CB_T)], rows_v)
        pltpu.sync_copy(coltc_h.at[pl.ds(base, CB_T)], tc_v)

        def it(jv, vsum):
            o = jv * 16
            m = tc_v[pl.ds(o, 16)]
            for r in range(NW):
                m = jnp.minimum(m, rows_v[r, pl.ds(o, 16)])
            return vsum + m

        vsum = lax.fori_loop(0, CB_T // 16, it,
                             jnp.zeros((16,), jnp.float32))
        total = jnp.sum(vsum)

        lane = lax.broadcasted_iota(jnp.int32, (16,), 0)
        out_v[...] = jnp.where(lane == 0, total, 0.0)
        pltpu.sync_copy(out_v, out_h.at[wid])

    return body(cols, coltc)


def kernel(pred, target):
    px, py, pz = pred[:, 0], pred[:, 1], pred[:, 2]
    tx, ty, tz = target[:, 0], target[:, 1], target[:, 2]
    qp = px * px + py * py + pz * pz
    qt = tx * tx + ty * ty + tz * tz

    tpad = TPAD - N
    tzp = jnp.zeros((tpad,), jnp.float32)
    t_x = jnp.concatenate([tx, tzp]).reshape(TROWS, 1, 128)
    t_y = jnp.concatenate([ty, tzp]).reshape(TROWS, 1, 128)
    t_z = jnp.concatenate([tz, tzp]).reshape(TROWS, 1, 128)
    t_q = jnp.concatenate([qt, jnp.full((tpad,), 1e30, jnp.float32)]
                          ).reshape(TROWS, 1, 128)
    s_x = (-2.0 * px[:MTC]).reshape(MTC // 8, 8, 1)
    s_y = (-2.0 * py[:MTC]).reshape(MTC // 8, 8, 1)
    s_z = (-2.0 * pz[:MTC]).reshape(MTC // 8, 8, 1)
    s_q = qp[:MTC].reshape(MTC // 8, 8, 1)

    rowsum_tc, colpart_tc = _tc_sweep(s_x, s_y, s_z, s_q, t_x, t_y, t_z, t_q)
    coltc_flat = colpart_tc.reshape(-1)

    spad = NSC_PAD - NSC
    szp = jnp.zeros((spad,), jnp.float32)
    sbp = jnp.full((spad,), 1e30, jnp.float32)
    src_x = jnp.concatenate([-2.0 * px[MTC:], szp])
    src_y = jnp.concatenate([-2.0 * py[MTC:], szp])
    src_z = jnp.concatenate([-2.0 * pz[MTC:], szp])
    src_q = jnp.concatenate([qp[MTC:], sbp])
    src_m = jnp.concatenate([jnp.ones((NSC,), jnp.float32), szp])

    rows_sc, cols_sc = _sc_sweep(src_x, src_y, src_z, src_q, src_m,
                                 tx, ty, tz, qt)
    colsums = _sc_combine(cols_sc, coltc_flat)
    return (rows_sc.sum() + rowsum_tc[0, 0] + colsums.sum()) / jnp.float32(N)


# speedup vs baseline: 6.5802x; 6.5802x over previous
import functools

import jax
import jax.numpy as jnp
from jax import lax
from jax.experimental import pallas as pl
from jax.experimental.pallas import tpu as pltpu
from jax.experimental.pallas import tpu_sc as plsc

N = 20000
TPAD = 20480
TROWS = TPAD // 128

MTC = 12800
SOCT = 4
GTC = MTC // (8 * SOCT)

NSC = N - MTC
NW = 32
SBLK = 8
NSC_PAD = ((NSC + NW * SBLK - 1) // (NW * SBLK)) * (NW * SBLK)
SRC_PER_W = NSC_PAD // NW
UNROLL = 2
NTV = N // 16
CB_T = 640
NCOL = TPAD


def _tc_sweep(sx, sy, sz, sq, tx, ty, tz, tq):
    def body(sx_r, sy_r, sz_r, sq_r, tx_r, ty_r, tz_r, tq_r,
             rowsum_r, colpart_r, colacc_r):
        i = pl.program_id(0)

        @pl.when(i == 0)
        def _():
            def ini(t, carry):
                colacc_r[:, pl.ds(t * 128, 128)] = jnp.full(
                    (8, 128), 3.0e38, jnp.float32)
                return carry
            lax.fori_loop(0, TROWS, ini, 0)

        sxo = [sx_r[o] for o in range(SOCT)]
        syo = [sy_r[o] for o in range(SOCT)]
        szo = [sz_r[o] for o in range(SOCT)]
        sqo = [sq_r[o] for o in range(SOCT)]

        rowinit = tuple(jnp.full((8, 128), 3.0e38, jnp.float32)
                        for _ in range(SOCT))

        def tile(t, rowaccs):
            rowaccs = list(rowaccs)
            txv = tx_r[t]
            tyv = ty_r[t]
            tzv = tz_r[t]
            tqv = tq_r[t]
            slab = colacc_r[:, pl.ds(t * 128, 128)]
            for o in range(SOCT):
                d = (tqv + txv * sxo[o] + tyv * syo[o]
                     + tzv * szo[o]) + sqo[o]
                rowaccs[o] = jnp.minimum(rowaccs[o], d)
                slab = jnp.minimum(slab, d)
            colacc_r[:, pl.ds(t * 128, 128)] = slab
            return tuple(rowaccs)

        rowaccs = lax.fori_loop(0, TROWS, tile, rowinit)

        blocksum = jnp.float32(0.0)
        for o in range(SOCT):
            blocksum = blocksum + jnp.sum(
                jnp.min(rowaccs[o], axis=1, keepdims=True))
        blocksum = blocksum.reshape(1, 1)

        @pl.when(i == 0)
        def _():
            rowsum_r[...] = blocksum

        @pl.when(i > 0)
        def _():
            rowsum_r[...] = rowsum_r[...] + blocksum

        @pl.when(i == GTC - 1)
        def _():
            def fin(t, carry):
                slab = colacc_r[:, pl.ds(t * 128, 128)]
                colpart_r[t] = jnp.min(slab, axis=0, keepdims=True)
                return carry
            lax.fori_loop(0, TROWS, fin, 0)

    return pl.pallas_call(
        body,
        grid=(GTC,),
        in_specs=[
            pl.BlockSpec((SOCT, 8, 128), lambda i: (i, 0, 0)),
            pl.BlockSpec((SOCT, 8, 128), lambda i: (i, 0, 0)),
            pl.BlockSpec((SOCT, 8, 128), lambda i: (i, 0, 0)),
            pl.BlockSpec((SOCT, 8, 128), lambda i: (i, 0, 0)),
            pl.BlockSpec((TROWS, 8, 128), lambda i: (0, 0, 0)),
            pl.BlockSpec((TROWS, 8, 128), lambda i: (0, 0, 0)),
            pl.BlockSpec((TROWS, 8, 128), lambda i: (0, 0, 0)),
            pl.BlockSpec((TROWS, 8, 128), lambda i: (0, 0, 0)),
        ],
        out_specs=[
            pl.BlockSpec((1, 1), lambda i: (0, 0)),
            pl.BlockSpec((TROWS, 1, 128), lambda i: (0, 0, 0)),
        ],
        out_shape=[
            jax.ShapeDtypeStruct((1, 1), jnp.float32),
            jax.ShapeDtypeStruct((TROWS, 1, 128), jnp.float32),
        ],
        scratch_shapes=[pltpu.VMEM((8, TPAD), jnp.float32)],
    )(sx, sy, sz, sq, tx, ty, tz, tq)


def _sc_sweep(src_x, src_y, src_z, src_q, src_m, tgt_x, tgt_y, tgt_z, tgt_q):
    mesh = plsc.VectorSubcoreMesh(core_axis_name="c", subcore_axis_name="s")

    @functools.partial(
        pl.kernel,
        mesh=mesh,
        out_type=(jax.ShapeDtypeStruct((NW, 16), jnp.float32),
                  jax.ShapeDtypeStruct((NW, NCOL), jnp.float32)),
        compiler_params=pltpu.CompilerParams(needs_layout_passes=False),
        scratch_types=[
            pltpu.VMEM((N,), jnp.float32),
            pltpu.VMEM((N,), jnp.float32),
            pltpu.VMEM((N,), jnp.float32),
            pltpu.VMEM((N,), jnp.float32),
            pltpu.VMEM((NCOL,), jnp.float32),
            pltpu.VMEM((SRC_PER_W + 16,), jnp.float32),
            pltpu.VMEM((SRC_PER_W + 16,), jnp.float32),
            pltpu.VMEM((SRC_PER_W + 16,), jnp.float32),
            pltpu.VMEM((SRC_PER_W + 16,), jnp.float32),
            pltpu.VMEM((SRC_PER_W + 16,), jnp.float32),
            pltpu.VMEM((16,), jnp.float32),
        ],
    )
    def body(src_x_h, src_y_h, src_z_h, src_q_h, src_m_h,
             tgt_x_h, tgt_y_h, tgt_z_h, tgt_q_h,
             rows_h, cols_h,
             tx_v, ty_v, tz_v, tq_v, col_v,
             sx_v, sy_v, sz_v, sq_v, sm_v, out_v):
        c = lax.axis_index("c")
        s = lax.axis_index("s")
        wid = c * 16 + s
        base = wid * SRC_PER_W

        pltpu.sync_copy(tgt_x_h, tx_v)
        pltpu.sync_copy(tgt_y_h, ty_v)
        pltpu.sync_copy(tgt_z_h, tz_v)
        pltpu.sync_copy(tgt_q_h, tq_v)
        pltpu.sync_copy(src_x_h.at[pl.ds(base, SRC_PER_W)],
                        sx_v.at[pl.ds(0, SRC_PER_W)])
        pltpu.sync_copy(src_y_h.at[pl.ds(base, SRC_PER_W)],
                        sy_v.at[pl.ds(0, SRC_PER_W)])
        pltpu.sync_copy(src_z_h.at[pl.ds(base, SRC_PER_W)],
                        sz_v.at[pl.ds(0, SRC_PER_W)])
        pltpu.sync_copy(src_q_h.at[pl.ds(base, SRC_PER_W)],
                        sq_v.at[pl.ds(0, SRC_PER_W)])
        pltpu.sync_copy(src_m_h.at[pl.ds(base, SRC_PER_W)],
                        sm_v.at[pl.ds(0, SRC_PER_W)])

        big = jnp.full((16,), 3.0e38, jnp.float32)

        def col_init(j, carry):
            col_v[pl.ds(j * 16, 16)] = big
            return carry

        lax.fori_loop(0, NTV, col_init, jnp.float32(0.0))

        zeros16 = jnp.zeros((16,), jnp.float32)

        def col_tail_init(j, carry):
            col_v[pl.ds(NTV * 16 + j * 16, 16)] = zeros16
            return carry

        lax.fori_loop(0, (NCOL - N) // 16, col_tail_init, jnp.float32(0.0))

        def src_block(b, psum):
            i0 = b * SBLK
            vx = sx_v[pl.ds(i0, 16)]
            vy = sy_v[pl.ds(i0, 16)]
            vz = sz_v[pl.ds(i0, 16)]
            vq = sq_v[pl.ds(i0, 16)]
            bx, by, bz, bq = [], [], [], []
            for k in range(SBLK):
                bx.append(jnp.full((16,), vx[k], jnp.float32))
                by.append(jnp.full((16,), vy[k], jnp.float32))
                bz.append(jnp.full((16,), vz[k], jnp.float32))
                bq.append(jnp.full((16,), vq[k], jnp.float32))

            init = tuple(big for _ in range(SBLK))

            def tgt_iter(j, accs):
                accs = list(accs)
                o = j * (16 * UNROLL)
                for u in range(UNROLL):
                    off = o + u * 16
                    tx = tx_v[pl.ds(off, 16)]
                    ty = ty_v[pl.ds(off, 16)]
                    tz = tz_v[pl.ds(off, 16)]
                    tq = tq_v[pl.ds(off, 16)]
                    cmin = None
                    for k in range(SBLK):
                        e = tq + bx[k] * tx + by[k] * ty + bz[k] * tz
                        accs[k] = jnp.minimum(accs[k], e)
                        d = e + bq[k]
                        cmin = d if cmin is None else jnp.minimum(cmin, d)
                    col_v[pl.ds(off, 16)] = jnp.minimum(
                        col_v[pl.ds(off, 16)], cmin)
                return tuple(accs)

            accs = lax.fori_loop(0, NTV // UNROLL, tgt_iter, init)

            mv = sm_v[pl.ds(i0, 16)]
            for k in range(SBLK):
                psum = psum + (jnp.min(accs[k]) + vq[k]) * mv[k]
            return psum

        psum = lax.fori_loop(0, SRC_PER_W // SBLK, src_block,
                             jnp.float32(0.0))

        lane = lax.broadcasted_iota(jnp.int32, (16,), 0)
        out_v[...] = jnp.where(lane == 0, psum, 0.0)
        pltpu.sync_copy(out_v, rows_h.at[wid])
        pltpu.sync_copy(col_v, cols_h.at[wid])

    return body(src_x, src_y, src_z, src_q, src_m,
                tgt_x, tgt_y, tgt_z, tgt_q)


def _sc_combine(cols, coltc):
    mesh = plsc.VectorSubcoreMesh(core_axis_name="c", subcore_axis_name="s")

    @functools.partial(
        pl.kernel,
        mesh=mesh,
        out_type=jax.ShapeDtypeStruct((NW, 16), jnp.float32),
        compiler_params=pltpu.CompilerParams(needs_layout_passes=False),
        scratch_types=[
            pltpu.VMEM((NW, CB_T), jnp.float32),
            pltpu.VMEM((CB_T,), jnp.float32),
            pltpu.VMEM((16,), jnp.float32),
        ],
    )
    def body(cols_h, coltc_h, out_h, rows_v, tc_v, out_v):
        c = lax.axis_index("c")
        s = lax.axis_index("s")
        wid = c * 16 + s
        base = wid * CB_T

        pltpu.sync_copy(cols_h.at[:, pl.ds(base, CB_T)], rows_v)
        pltpu.sync_copy(coltc_h.at[pl.ds(base, CB_T)], tc_v)

        def it(jv, vsum):
            o = jv * 16
            m = tc_v[pl.ds(o, 16)]
            for r in range(NW):
                m = jnp.minimum(m, rows_v[r, pl.ds(o, 16)])
            return vsum + m

        vsum = lax.fori_loop(0, CB_T // 16, it,
                             jnp.zeros((16,), jnp.float32))
        total = jnp.sum(vsum)

        lane = lax.broadcasted_iota(jnp.int32, (16,), 0)
        out_v[...] = jnp.where(lane == 0, total, 0.0)
        pltpu.sync_copy(out_v, out_h.at[wid])

    return body(cols, coltc)


def kernel(pred, target):
    px, py, pz = pred[:, 0], pred[:, 1], pred[:, 2]
    tx, ty, tz = target[:, 0], target[:, 1], target[:, 2]
    qp = px * px + py * py + pz * pz
    qt = tx * tx + ty * ty + tz * tz

    tpad = TPAD - N
    tzp = jnp.zeros((tpad,), jnp.float32)

    def t8(a):
        return jnp.broadcast_to(a.reshape(TROWS, 1, 128), (TROWS, 8, 128))

    def s8(a):
        return jnp.broadcast_to(a.reshape(MTC // 8, 8, 1), (MTC // 8, 8, 128))

    t_x = t8(jnp.concatenate([tx, tzp]))
    t_y = t8(jnp.concatenate([ty, tzp]))
    t_z = t8(jnp.concatenate([tz, tzp]))
    t_q = t8(jnp.concatenate([qt, jnp.full((tpad,), 1e30, jnp.float32)]))
    s_x = s8(-2.0 * px[:MTC])
    s_y = s8(-2.0 * py[:MTC])
    s_z = s8(-2.0 * pz[:MTC])
    s_q = s8(qp[:MTC])

    rowsum_tc, colpart_tc = _tc_sweep(s_x, s_y, s_z, s_q, t_x, t_y, t_z, t_q)
    coltc_flat = colpart_tc.reshape(-1)

    spad = NSC_PAD - NSC
    szp = jnp.zeros((spad,), jnp.float32)
    sbp = jnp.full((spad,), 1e30, jnp.float32)
    src_x = jnp.concatenate([-2.0 * px[MTC:], szp])
    src_y = jnp.concatenate([-2.0 * py[MTC:], szp])
    src_z = jnp.concatenate([-2.0 * pz[MTC:], szp])
    src_q = jnp.concatenate([qp[MTC:], sbp])
    src_m = jnp.concatenate([jnp.ones((NSC,), jnp.float32), szp])

    rows_sc, cols_sc = _sc_sweep(src_x, src_y, src_z, src_q, src_m,
                                 tx, ty, tz, qt)
    colsums = _sc_combine(cols_sc, coltc_flat)
    return (rows_sc.sum() + rowsum_tc[0, 0] + colsums.sum()) / jnp.float32(N)
